# baseline (speedup 1.0000x reference)
import functools

import jax
import jax.numpy as jnp
from jax import lax
from jax.experimental import pallas as pl
from jax.experimental.pallas import tpu as pltpu
from jax.experimental.pallas import tpu_sc as plsc

N = 10000
E = 320000
D = 128
NC = 2
NS = 16
NW = NC * NS
CH = 128
EPT = E // NW
NCHT = 80
EPTP = NCHT * CH
NACC = 10016
ZR = 48
RPS = 624
TAIL = NACC - NS * RPS


def _sc_spmm(features, edata):
    mesh = plsc.VectorSubcoreMesh(core_axis_name="c", subcore_axis_name="s")

    @functools.partial(
        pl.kernel,
        out_type=jax.ShapeDtypeStruct((NC, NACC, D), jnp.float32),
        mesh=mesh,
        scratch_types=[
            pltpu.VMEM((3, CH), jnp.int32),
            pltpu.VMEM((3, CH), jnp.int32),
            pltpu.VMEM((CH,), jnp.int32),
            pltpu.VMEM((CH,), jnp.int32),
            pltpu.VMEM((CH, D), jnp.float32),
            pltpu.VMEM((CH, D), jnp.float32),
            pltpu.VMEM((ZR, D), jnp.float32),
            pltpu.VMEM_SHARED((NACC, D), jnp.float32),
            pltpu.SemaphoreType.DMA,
            pltpu.SemaphoreType.DMA,
            pltpu.SemaphoreType.DMA,
            pltpu.SemaphoreType.DMA,
            pltpu.SemaphoreType.DMA,
            pltpu.SemaphoreType.DMA,
        ],
        compiler_params=pltpu.CompilerParams(needs_layout_passes=False),
    )
    def k(feat_hbm, edata_hbm, out_hbm,
          ed0, ed1, dc0, dc1, rows0, rows1, zbuf, acc,
          g0, g1, s0, s1, e0, e1):
        c = lax.axis_index("c")
        s = lax.axis_index("s")
        wid = s * NC + c
        eds = (ed0, ed1)
        dstc = (dc0, dc1)
        rows = (rows0, rows1)
        gsem = (g0, g1)
        ssem = (s0, s1)
        esem = (e0, e1)
        base_g = wid * NCHT

        def zero_row(r, _):
            for d in range(D // 16):
                zbuf[r, pl.ds(d * 16, 16)] = jnp.zeros((16,), jnp.float32)
            return _
        lax.fori_loop(0, ZR, zero_row, None)
        for j in range(RPS // ZR):
            pltpu.sync_copy(zbuf, acc.at[pl.ds(s * RPS + j * ZR, ZR)])

        @pl.when(s == 0)
        def _():
            pltpu.sync_copy(zbuf.at[pl.ds(0, TAIL)],
                            acc.at[pl.ds(NS * RPS, TAIL)])
        plsc.subcore_barrier()

        def issue_edata(i, p):
            pltpu.async_copy(edata_hbm.at[base_g + i], eds[p], esem[p])

        def wait_edata(i, p):
            pltpu.make_async_copy(edata_hbm.at[base_g + i], eds[p],
                                  esem[p]).wait()

        def issue_gather(p):
            pltpu.async_copy(feat_hbm.at[eds[p].at[0]], rows[p], gsem[p])

        def wait_gather(p):
            pltpu.make_async_copy(feat_hbm.at[eds[p].at[0]], rows[p],
                                  gsem[p]).wait()

        def copy_dst(p):
            for r in range(CH // 16):
                sl = pl.ds(r * 16, 16)
                dstc[p][sl] = eds[p][1, sl]

        def issue_scatter(p):
            pltpu.async_copy(rows[p], acc.at[dstc[p]], ssem[p], add=True)

        def wait_scatter(p):
            pltpu.make_async_copy(rows[p], acc.at[dstc[p]], ssem[p]).wait()

        def scale(p):
            @plsc.parallel_loop(0, CH, unroll=8)
            def _(e):
                vi = plsc.load_gather(eds[p].at[2],
                                      [jnp.full((16,), e, jnp.int32)])
                vv = plsc.bitcast(vi, jnp.float32)
                for d in range(D // 16):
                    sl = pl.ds(d * 16, 16)
                    rows[p][e, sl] = rows[p][e, sl] * vv

        issue_edata(0, 0)
        issue_edata(1, 1)
        wait_edata(0, 0)
        issue_gather(0)
        wait_gather(0)
        copy_dst(0)
        wait_edata(1, 1)
        issue_gather(1)
        scale(0)
        issue_scatter(0)
        issue_edata(2, 0)
        wait_gather(1)
        copy_dst(1)
        wait_scatter(0)
        wait_edata(2, 0)
        issue_gather(0)
        scale(1)
        issue_scatter(1)
        issue_edata(3, 1)

        def step(i, p):
            wait_gather(p)
            copy_dst(p)
            wait_scatter(1 - p)

            @pl.when(i + 1 < NCHT)
            def _():
                wait_edata(i + 1, 1 - p)
                issue_gather(1 - p)
            scale(p)
            issue_scatter(p)

            @pl.when(i + 2 < NCHT)
            def _():
                issue_edata(i + 2, p)

        def pair(i2, _):
            i = 2 + 2 * i2
            step(i, 0)
            step(i + 1, 1)
            return _
        lax.fori_loop(0, (NCHT - 2) // 2, pair, None)

        wait_scatter(1)

        plsc.subcore_barrier()
        for j in range(RPS // ZR):
            off = s * RPS + j * ZR
            pltpu.sync_copy(acc.at[pl.ds(off, ZR)],
                            out_hbm.at[c].at[pl.ds(off, ZR)])

        @pl.when(s == 0)
        def _():
            pltpu.sync_copy(acc.at[pl.ds(NS * RPS, TAIL)],
                            out_hbm.at[c].at[pl.ds(NS * RPS, TAIL)])

    return k(features, edata)


def _tc_combine(features, partial, W1, b1, W2, b2):
    BN = 1000
    bias = (b1 + b2).reshape(1, D)
    p0 = partial[0, :N]
    p1 = partial[1, :N]

    def body(f_ref, p0_ref, p1_ref, w1_ref, w2_ref, b_ref, o_ref):
        lap = p0_ref[...] + p1_ref[...]
        f = f_ref[...]
        m1 = lap + f
        m2 = lap * f
        dn = (((1,), (1,)), ((), ()))
        o_ref[...] = (
            lax.dot_general(m1, w1_ref[...], dn,
                            preferred_element_type=jnp.float32)
            + lax.dot_general(m2, w2_ref[...], dn,
                              preferred_element_type=jnp.float32)
            + b_ref[...]
        )

    row_spec = pl.BlockSpec((BN, D), lambda i: (i, 0))
    full_spec = pl.BlockSpec((D, D), lambda i: (0, 0))
    return pl.pallas_call(
        body,
        grid=(N // BN,),
        in_specs=[row_spec, row_spec, row_spec, full_spec, full_spec,
                  pl.BlockSpec((1, D), lambda i: (0, 0))],
        out_specs=row_spec,
        out_shape=jax.ShapeDtypeStruct((N, D), jnp.float32),
    )(features, p0, p1, W1, W2, bias)


@jax.jit
def kernel(features, edge_index, edge_vals, W1, b1, W2, b2):
    dst = edge_index[0]
    src = edge_index[1]
    vals_i = lax.bitcast_convert_type(edge_vals, jnp.int32)
    pad = EPTP - EPT
    src_p = jnp.pad(src.reshape(NW, EPT), ((0, 0), (0, pad)))
    dst_p = jnp.pad(dst.reshape(NW, EPT), ((0, 0), (0, pad)),
                    constant_values=N)
    val_p = jnp.pad(vals_i.reshape(NW, EPT), ((0, 0), (0, pad)))
    edata = jnp.stack([src_p.reshape(NW * NCHT, CH),
                       dst_p.reshape(NW * NCHT, CH),
                       val_p.reshape(NW * NCHT, CH)], axis=1)
    partial = _sc_spmm(features, edata)
    return _tc_combine(features, partial, W1, b1, W2, b2)
